# baseline (speedup 1.0000x reference)
import functools

import jax
import jax.numpy as jnp
from jax import lax
from jax.experimental import pallas as pl
from jax.experimental.pallas import tpu as pltpu
from jax.experimental.pallas import tpu_sc as plsc

BATCH = 16384
DEPTH = 1000
NUM_CORES = 2
NUM_SUBCORES = 16
NUM_WORKERS = NUM_CORES * NUM_SUBCORES
PER_W = BATCH // NUM_WORKERS
CW = 128
CHUNKS = PER_W // CW
LANES = 16
GROUPS = CW // LANES
QB = (0, 248, 504, 752, 1000)
NQ = len(QB) - 1


def _one_hot_body(idx_hbm, out_hbm, idx_v, b0, b1, b2, b3, sem, isem):
    wid = lax.axis_index("s") * NUM_CORES + lax.axis_index("c")
    base = wid * PER_W
    bufs = (b0, b1, b2, b3)

    idx_cp = pltpu.async_copy(
        idx_hbm.at[pl.ds(base * 1, PER_W)], idx_v, isem
    )

    zeros16 = jnp.zeros((LANES,), jnp.float32)
    ones16 = jnp.full((LANES,), 1.0, jnp.float32)
    iota16 = lax.iota(jnp.int32, LANES)

    def scatter_q(c, q, vals):
        lo, hi = QB[q], QB[q + 1]
        buf = bufs[q]
        for g in range(GROUPS):
            idx16 = idx_v[pl.ds(c * CW + g * LANES, LANES)]
            cols16 = g * LANES + iota16
            rel16 = jnp.minimum(jnp.maximum(idx16 - lo, 0), hi - lo - 1)
            in_q = (idx16 >= lo) & (idx16 < hi)
            v = jnp.where(in_q, vals, zeros16)
            plsc.store_scatter(buf, [rel16, cols16], v)

    def dma_q(c, q):
        lo, hi = QB[q], QB[q + 1]
        return pltpu.async_copy(
            bufs[q],
            out_hbm.at[pl.ds(lo, hi - lo), pl.ds(base + c * CW, CW)],
            sem,
        )

    handles = [None] * NQ
    for c in range(CHUNKS):
        for q in range(NQ):
            if handles[q] is None:
                nrows = QB[q + 1] - QB[q]
                buf = bufs[q]

                def _zero(r, _, buf=buf):
                    for u in range(GROUPS):
                        buf[r, pl.ds(u * LANES, LANES)] = zeros16
                    return _

                lax.fori_loop(0, nrows, _zero, None)
                if q == 0:
                    idx_cp.wait()
            else:
                handles[q].wait()
                scatter_q(c - 1, q, zeros16)
            scatter_q(c, q, ones16)
            handles[q] = dma_q(c, q)
    for q in range(NQ):
        handles[q].wait()


@functools.partial(jax.jit, static_argnames=())
def _one_hot_sc(idx):
    mesh = plsc.VectorSubcoreMesh(core_axis_name="c", subcore_axis_name="s")
    k = functools.partial(
        pl.kernel,
        mesh=mesh,
        out_type=jax.ShapeDtypeStruct((DEPTH, BATCH), jnp.float32),
        scratch_types=[
            pltpu.VMEM((PER_W,), jnp.int32),
            pltpu.VMEM((QB[1] - QB[0], CW), jnp.float32),
            pltpu.VMEM((QB[2] - QB[1], CW), jnp.float32),
            pltpu.VMEM((QB[3] - QB[2], CW), jnp.float32),
            pltpu.VMEM((QB[4] - QB[3], CW), jnp.float32),
            pltpu.SemaphoreType.DMA,
            pltpu.SemaphoreType.DMA,
        ],
        compiler_params=pltpu.CompilerParams(
            needs_layout_passes=False,
            use_tc_tiling_on_sc=True,
        ),
    )(_one_hot_body)
    return k(idx).T


def kernel(X_in, ones):
    del ones
    return _one_hot_sc(X_in.astype(jnp.int32))


# speedup vs baseline: 1.0928x; 1.0928x over previous
import functools

import jax
import jax.numpy as jnp
from jax import lax
from jax.experimental import pallas as pl
from jax.experimental.pallas import tpu as pltpu
from jax.experimental.pallas import tpu_sc as plsc

BATCH = 16384
DEPTH = 1000
NUM_CORES = 2
NUM_SUBCORES = 16
NUM_WORKERS = NUM_CORES * NUM_SUBCORES
PER_W = BATCH // NUM_WORKERS
CW = 128
CHUNKS = PER_W // CW
LANES = 16
GROUPS = CW // LANES
QBOUNDS = (0, 248, 504, 752, 1000)


def _one_hot_body(idx_hbm, out_hbm, idx_v, buf, sem, isem):
    wid = lax.axis_index("s") * NUM_CORES + lax.axis_index("c")
    base = wid * PER_W

    idx_cp = pltpu.async_copy(
        idx_hbm.at[pl.ds(base * 1, PER_W)], idx_v, isem
    )

    zeros16 = jnp.zeros((LANES,), jnp.float32)
    ones16 = jnp.full((LANES,), 1.0, jnp.float32)
    iota16 = lax.iota(jnp.int32, LANES)

    def _zero(r, _):
        for u in range(GROUPS):
            buf[r, pl.ds(u * LANES, LANES)] = zeros16
        return _

    def scatter_stripe(c, vals):
        for g in range(GROUPS):
            idx16 = idx_v[pl.ds(c * CW + g * LANES, LANES)]
            cols16 = g * LANES + iota16
            plsc.store_scatter(buf, [idx16, cols16], vals)

    def scatter_range(c, lo, hi):
        for g in range(GROUPS):
            idx16 = idx_v[pl.ds(c * CW + g * LANES, LANES)]
            cols16 = g * LANES + iota16
            in_range = (idx16 >= lo) & (idx16 < hi)
            vals = jnp.where(in_range, ones16, zeros16)
            plsc.store_scatter(buf, [idx16, cols16], vals)

    def stripe_dma(c, lo, nrows):
        return pltpu.async_copy(
            buf.at[pl.ds(lo, nrows)],
            out_hbm.at[pl.ds(lo, nrows), pl.ds(base + c * CW, CW)],
            sem,
        )

    lax.fori_loop(0, QBOUNDS[1], _zero, None)
    idx_cp.wait()
    prev = None
    for q in range(len(QBOUNDS) - 1):
        lo, hi = QBOUNDS[q], QBOUNDS[q + 1]
        if prev is not None:
            prev.wait()
        scatter_range(0, lo, hi)
        prev = stripe_dma(0, lo, hi - lo)
        if q + 2 < len(QBOUNDS):
            lax.fori_loop(QBOUNDS[q + 1], QBOUNDS[q + 2], _zero, None)
    prev.wait()
    scatter_stripe(0, zeros16)

    for c in range(1, CHUNKS):
        scatter_stripe(c, ones16)
        stripe_dma(c, 0, DEPTH).wait()
        if c + 1 < CHUNKS:
            scatter_stripe(c, zeros16)


@functools.partial(jax.jit, static_argnames=())
def _one_hot_sc(idx):
    mesh = plsc.VectorSubcoreMesh(core_axis_name="c", subcore_axis_name="s")
    k = functools.partial(
        pl.kernel,
        mesh=mesh,
        out_type=jax.ShapeDtypeStruct((DEPTH, BATCH), jnp.float32),
        scratch_types=[
            pltpu.VMEM((PER_W,), jnp.int32),
            pltpu.VMEM((DEPTH, CW), jnp.float32),
            pltpu.SemaphoreType.DMA,
            pltpu.SemaphoreType.DMA,
        ],
        compiler_params=pltpu.CompilerParams(
            needs_layout_passes=False,
            use_tc_tiling_on_sc=True,
        ),
    )(_one_hot_body)
    return k(idx).T


def kernel(X_in, ones):
    del ones
    return _one_hot_sc(X_in.astype(jnp.int32))
